# baseline (speedup 1.0000x reference)
import functools

import jax
import jax.numpy as jnp
from jax import lax
from jax.experimental import pallas as pl
from jax.experimental.pallas import tpu as pltpu
from jax.experimental.pallas import tpu_sc as plsc

B, S, D = 16, 512, 768
NL = 4
SW = D
NV = SW // 16
NROWS = B * S
REG = 544
K = 8
IDXLEN = B * REG + 64
LANES = 16
NSEG = 2 * B
NW = 32
SB = 4
NSEGP = 2 * SB
NCHMAX = B * (REG // K + 2) + 8 * NW + 8

_mesh = plsc.VectorSubcoreMesh(core_axis_name="c", subcore_axis_name="s")


def _al8(x):
    return pl.multiple_of(x, 8)


def _lane_iota():
    return jnp.arange(LANES, dtype=jnp.int32)


def _p1_body(table, ids_h, am_h, typ_h, part_h, cnt_h,
             row_i, row_a, row_t, bld, msk, idxb, gbs, parts, cntv,
             cstage, wl, sh_idx, sh_cnt, sem0):
    ci = lax.axis_index("c")
    si = lax.axis_index("s")
    b = si
    iota = _lane_iota()
    zero16 = jnp.zeros((LANES,), jnp.int32)
    zf = jnp.zeros((LANES,), jnp.float32)

    pltpu.sync_copy(ids_h.at[b], row_i)
    pltpu.sync_copy(am_h.at[b], row_a)
    pltpu.sync_copy(typ_h.at[b], row_t)

    def _zb(i, _):
        bld[pl.ds(_al8(i * LANES), LANES)] = zero16
        return 0
    lax.fori_loop(0, (REG + LANES) // LANES, _zb, 0)

    c_b = b * S

    def _pass1(v, carry):
        zcarry, off, nt = carry
        o = _al8(v * LANES)
        iv = row_i[pl.ds(o, LANES)]
        av = row_a[pl.ds(o, LANES)]
        tv = row_t[pl.ds(o, LANES)]
        z = (iv == 0).astype(jnp.int32)
        cs = jnp.cumsum(z) + zcarry
        valid = (cs == 0) & (iv != 101) & (iv != 102) & (av == 1)
        mterm = valid & (tv == 0)
        mtext = valid & (tv == 1)
        msk[pl.ds(o, LANES)] = mtext.astype(jnp.int32)
        pos = iota + v * LANES + c_b
        mi = mterm.astype(jnp.int32)
        h = jnp.sum(mi)
        pr = jnp.cumsum(mi) - mi + off
        plsc.store_scatter(bld, [pr], pos, mask=mterm)
        return zcarry + jnp.sum(z), off + h, nt + h

    _, off, nt = lax.fori_loop(0, S // LANES, _pass1, (0, 0, 0))
    off = (off + K - 1) & (-K)

    def _pass2(v, carry):
        off, nx = carry
        o = _al8(v * LANES)
        mtext = msk[pl.ds(o, LANES)] != 0
        pos = iota + v * LANES + c_b
        mi = mtext.astype(jnp.int32)
        h = jnp.sum(mi)
        pr = jnp.cumsum(mi) - mi + off
        plsc.store_scatter(bld, [pr], pos, mask=mtext)
        return off + h, nx + h

    _, nx = lax.fori_loop(0, S // LANES, _pass2, (off, 0))

    cstage[...] = (jnp.where(iota == 0, nt, 0)
                   + jnp.where(iota == 1, nx, 0))
    pltpu.sync_copy(cstage, sh_cnt.at[b])
    pltpu.sync_copy(bld.at[pl.ds(0, REG)], sh_idx.at[b])

    @pl.when(ci == 0)
    def _():
        pltpu.sync_copy(cstage, cnt_h.at[b])

    plsc.subcore_barrier()

    pltpu.sync_copy(sh_cnt, cntv)
    for bb in range(SB):
        pltpu.sync_copy(sh_idx.at[bb], idxb.at[pl.ds(bb * REG, REG)])
    for i in range(64 // LANES):
        idxb[pl.ds(B * REG + i * LANES, LANES)] = zero16

    w = ci * 16 + si

    def _zp(i, _):
        parts[pl.ds(_al8(i * LANES), LANES)] = zf
        return 0
    lax.fori_loop(0, NSEGP * SW // LANES, _zp, 0)

    cc = 0
    for bb in range(SB):
        row = cntv[bb]
        ntb = row[0]
        nxb = row[1]
        t1s = (ntb + K - 1) & (-K)
        for t in range(2):
            stt = bb * REG + t1s * t
            ln = jnp.where(t == 0, ntb, nxb)
            seg = 2 * bb + t
            nch = (ln + K - 1) // K

            def _wl(j, cc, stt=stt, ln=ln, seg=seg):
                word = ((stt + j * K) | (seg << 15)
                        | (jnp.minimum(ln - j * K, 15) << 20))
                wl[cc] = word
                return cc + 1
            cc = lax.fori_loop(0, nch, _wl, cc)
    def _snt(i, _):
        wl[cc + i] = 0
        return 0
    lax.fori_loop(0, 8 * NW, _snt, 0)

    def _issue(word, buf, sem):
        o = pl.multiple_of(word & 32767, 8)
        return pltpu.async_copy(table.at[idxb.at[pl.ds(o, K)]], buf, sem)

    def _fold_into_parts(a, seg):
        for k in range(NV):
            plsc.addupdate(parts.at[pl.ds(_al8(seg * SW + k * LANES), LANES)],
                           a[k])

    def _acc_buf(buf, word):
        lnr = (word >> 20) & 15
        seg = (word >> 15) & 31
        a = [zf] * NV
        for j in range(K):
            c = j < lnr
            for k in range(NV):
                r = buf[j, pl.ds(k * LANES, LANES)]
                a[k] = a[k] + jnp.where(c, r, zf)
        _fold_into_parts(a, seg)

    q = (((cc + NW - 1) // NW) + 1) & (-2)
    base = w * q
    n2 = q // 2

    def _duo(i2, _):
        b2 = base + i2 * 2
        cps = []
        for u in range(2):
            cps.append(_issue(wl[b2 + u], gbs.at[u], sem0))
        for u in range(2):
            cps[u].wait()
            _acc_buf(gbs.at[u], wl[b2 + u])
        return 0

    lax.fori_loop(0, n2, _duo, 0)

    woff = _al8(w * (NSEGP * SW))
    pltpu.sync_copy(parts, part_h.at[pl.ds(woff, NSEGP * SW)])


NLT = NL - 1
NSTEP = B * NLT + (B - SB)


def _bl_of(i):
    lo = i < SB * NLT
    j = i - SB * NLT
    bv = jnp.where(lo, i // NLT, SB + j // (NLT + 1))
    lv = jnp.where(lo, i % NLT, j % (NLT + 1))
    return bv, lv


def _tc_body(hs_ref, ids_ref, am_ref, typ_ref, out_ref):
    _, li = _bl_of(pl.program_id(0))
    ids = ids_ref[0]
    am = am_ref[0]
    typ = typ_ref[0]
    z = (ids == 0).astype(jnp.float32)
    tri = (lax.broadcasted_iota(jnp.int32, (S, S), 0)
           <= lax.broadcasted_iota(jnp.int32, (S, S), 1)).astype(jnp.float32)
    cz = jnp.dot(z, tri, preferred_element_type=jnp.float32)
    valid = (cz == 0) & (ids != 101) & (ids != 102) & (am == 1)
    mterm = (valid & (typ == 0)).astype(jnp.float32)
    mtext = (valid & (typ == 1)).astype(jnp.float32)
    hsb = hs_ref[0, 0]
    wt = jnp.sum(hsb * mterm.reshape(S, 1), axis=0)
    wx = jnp.sum(hsb * mtext.reshape(S, 1), axis=0)
    part = jnp.concatenate([wt[None, :], wx[None, :]], axis=0)

    @pl.when(li == 0)
    def _():
        out_ref[0] = part

    @pl.when(li > 0)
    def _():
        out_ref[0] = out_ref[0] + part


def _hs_map(i):
    bv, lv = _bl_of(i)
    return (lv, bv, 0, 0)


def _row_map(i):
    bv, _ = _bl_of(i)
    return (bv, 0, 0)


_tc_sums = pl.pallas_call(
    _tc_body,
    grid=(NSTEP,),
    in_specs=[
        pl.BlockSpec((1, 1, S, D), _hs_map),
        pl.BlockSpec((1, 1, S), _row_map),
        pl.BlockSpec((1, 1, S), _row_map),
        pl.BlockSpec((1, 1, S), _row_map),
    ],
    out_specs=pl.BlockSpec((1, 2, D), _row_map),
    out_shape=jax.ShapeDtypeStruct((B, 2, D), jnp.float32),
    compiler_params=pltpu.CompilerParams(
        dimension_semantics=("arbitrary",)),
)


def _combine_body(parts_ref, tc_ref, n2_ref, out_ref):
    psum = jnp.sum(parts_ref[...], axis=0)
    pad = jnp.zeros((2 * B - NSEGP, SW), jnp.float32)
    total = tc_ref[...] + jnp.concatenate([psum, pad], axis=0)
    out_ref[...] = total * (jnp.float32(0.25) / n2_ref[...])


_combine = pl.pallas_call(
    _combine_body,
    out_shape=jax.ShapeDtypeStruct((2 * B, D), jnp.float32),
)

_phase1 = functools.partial(
    pl.kernel,
    out_type=[jax.ShapeDtypeStruct((NW * NSEGP * SW,), jnp.float32),
              jax.ShapeDtypeStruct((B, LANES), jnp.int32)],
    mesh=_mesh,
    compiler_params=pltpu.CompilerParams(needs_layout_passes=False,
                                         use_tc_tiling_on_sc=False),
    scratch_types=[
        pltpu.VMEM((S,), jnp.int32),
        pltpu.VMEM((S,), jnp.int32),
        pltpu.VMEM((S,), jnp.int32),
        pltpu.VMEM((REG + LANES,), jnp.int32),
        pltpu.VMEM((S,), jnp.int32),
        pltpu.VMEM((IDXLEN,), jnp.int32),
        pltpu.VMEM((2, K, SW), jnp.float32),
        pltpu.VMEM((NSEGP * SW,), jnp.float32),
        pltpu.VMEM((B, LANES), jnp.int32),
        pltpu.VMEM((LANES,), jnp.int32),
        pltpu.SMEM((NCHMAX,), jnp.int32),
        pltpu.VMEM_SHARED((B, REG), jnp.int32),
        pltpu.VMEM_SHARED((B, LANES), jnp.int32),
        pltpu.SemaphoreType.DMA,
    ],
)(_p1_body)

@jax.jit
def kernel(bert_out, input_ids, attention_mask, token_type_ids):
    table = bert_out[-1].reshape(NROWS, SW)
    hs4 = bert_out[-NL:]
    ids32 = input_ids.astype(jnp.int32)
    am32 = attention_mask.astype(jnp.int32)
    typ32 = token_type_ids.astype(jnp.int32)
    part, cnt = _phase1(table, ids32, am32, typ32)
    tc = _tc_sums(hs4, ids32.reshape(B, 1, S), am32.reshape(B, 1, S),
                  typ32.reshape(B, 1, S))
    n2 = cnt[:, 0:2].reshape(2 * B, 1).astype(jnp.float32)
    out2 = _combine(part.reshape(NW, NSEGP, SW), tc.reshape(2 * B, D), n2)
    return out2.reshape(B, 2 * D)


# speedup vs baseline: 1.1979x; 1.1979x over previous
import functools

import jax
import jax.numpy as jnp
from jax import lax
from jax.experimental import pallas as pl
from jax.experimental.pallas import tpu as pltpu
from jax.experimental.pallas import tpu_sc as plsc

B, S, D = 16, 512, 768
NL = 4
SW = D
NV = SW // 16
NROWS = B * S
REG = 544
K = 8
IDXLEN = B * REG + 64
LANES = 16
NSEG = 2 * B
NW = 16
SB = 4
NSEGP = 2 * SB
NCHMAX = B * (REG // K + 2) + 8 * NW + 8

_mesh = plsc.VectorSubcoreMesh(core_axis_name="c", subcore_axis_name="s",
                               num_cores=1)


def _al8(x):
    return pl.multiple_of(x, 8)


def _lane_iota():
    return jnp.arange(LANES, dtype=jnp.int32)


def _p1_body(table, ids_h, am_h, typ_h, part_h, cnt_h,
             row_i, row_a, row_t, bld, msk, idxb, gbs, parts, cntv,
             cstage, wl, sh_idx, sh_cnt, sem0):
    ci = lax.axis_index("c")
    si = lax.axis_index("s")
    b = si
    iota = _lane_iota()
    zero16 = jnp.zeros((LANES,), jnp.int32)
    zf = jnp.zeros((LANES,), jnp.float32)

    pltpu.sync_copy(ids_h.at[b], row_i)
    pltpu.sync_copy(am_h.at[b], row_a)
    pltpu.sync_copy(typ_h.at[b], row_t)

    def _zb(i, _):
        bld[pl.ds(_al8(i * LANES), LANES)] = zero16
        return 0
    lax.fori_loop(0, (REG + LANES) // LANES, _zb, 0)

    c_b = b * S

    def _pass1(v, carry):
        zcarry, off, nt = carry
        o = _al8(v * LANES)
        iv = row_i[pl.ds(o, LANES)]
        av = row_a[pl.ds(o, LANES)]
        tv = row_t[pl.ds(o, LANES)]
        z = (iv == 0).astype(jnp.int32)
        cs = jnp.cumsum(z) + zcarry
        valid = (cs == 0) & (iv != 101) & (iv != 102) & (av == 1)
        mterm = valid & (tv == 0)
        mtext = valid & (tv == 1)
        msk[pl.ds(o, LANES)] = mtext.astype(jnp.int32)
        pos = iota + v * LANES + c_b
        mi = mterm.astype(jnp.int32)
        h = jnp.sum(mi)
        pr = jnp.cumsum(mi) - mi + off
        plsc.store_scatter(bld, [pr], pos, mask=mterm)
        return zcarry + jnp.sum(z), off + h, nt + h

    _, off, nt = lax.fori_loop(0, S // LANES, _pass1, (0, 0, 0))
    off = (off + K - 1) & (-K)

    def _pass2(v, carry):
        off, nx = carry
        o = _al8(v * LANES)
        mtext = msk[pl.ds(o, LANES)] != 0
        pos = iota + v * LANES + c_b
        mi = mtext.astype(jnp.int32)
        h = jnp.sum(mi)
        pr = jnp.cumsum(mi) - mi + off
        plsc.store_scatter(bld, [pr], pos, mask=mtext)
        return off + h, nx + h

    _, nx = lax.fori_loop(0, S // LANES, _pass2, (off, 0))

    cstage[...] = (jnp.where(iota == 0, nt, 0)
                   + jnp.where(iota == 1, nx, 0))
    pltpu.sync_copy(cstage, sh_cnt.at[b])
    pltpu.sync_copy(bld.at[pl.ds(0, REG)], sh_idx.at[b])

    @pl.when(ci == 0)
    def _():
        pltpu.sync_copy(cstage, cnt_h.at[b])

    plsc.subcore_barrier()

    pltpu.sync_copy(sh_cnt, cntv)
    for bb in range(SB):
        pltpu.sync_copy(sh_idx.at[bb], idxb.at[pl.ds(bb * REG, REG)])
    for i in range(64 // LANES):
        idxb[pl.ds(B * REG + i * LANES, LANES)] = zero16

    w = ci * 16 + si

    def _zp(i, _):
        parts[pl.ds(_al8(i * LANES), LANES)] = zf
        return 0
    lax.fori_loop(0, NSEGP * SW // LANES, _zp, 0)

    cc = 0
    for bb in range(SB):
        row = cntv[bb]
        ntb = row[0]
        nxb = row[1]
        t1s = (ntb + K - 1) & (-K)
        for t in range(2):
            stt = bb * REG + t1s * t
            ln = jnp.where(t == 0, ntb, nxb)
            seg = 2 * bb + t
            nch = (ln + K - 1) // K

            def _wl(j, cc, stt=stt, ln=ln, seg=seg):
                word = ((stt + j * K) | (seg << 15)
                        | (jnp.minimum(ln - j * K, 15) << 20))
                wl[cc] = word
                return cc + 1
            cc = lax.fori_loop(0, nch, _wl, cc)
    def _snt(i, _):
        wl[cc + i] = 0
        return 0
    lax.fori_loop(0, 8 * NW, _snt, 0)

    def _issue(word, buf, sem):
        o = pl.multiple_of(word & 32767, 8)
        return pltpu.async_copy(table.at[idxb.at[pl.ds(o, K)]], buf, sem)

    def _fold_into_parts(a, seg):
        for k in range(NV):
            plsc.addupdate(parts.at[pl.ds(_al8(seg * SW + k * LANES), LANES)],
                           a[k])

    def _acc_buf(buf, word):
        lnr = (word >> 20) & 15
        seg = (word >> 15) & 31
        a = [zf] * NV
        for j in range(K):
            c = j < lnr
            for k in range(NV):
                r = buf[j, pl.ds(k * LANES, LANES)]
                a[k] = a[k] + jnp.where(c, r, zf)
        _fold_into_parts(a, seg)

    q = (((cc + NW - 1) // NW) + 1) & (-2)
    base = w * q
    n2 = q // 2

    def _duo(i2, _):
        b2 = base + i2 * 2
        cps = []
        for u in range(2):
            cps.append(_issue(wl[b2 + u], gbs.at[u], sem0))
        for u in range(2):
            cps[u].wait()
            _acc_buf(gbs.at[u], wl[b2 + u])
        return 0

    lax.fori_loop(0, n2, _duo, 0)

    woff = _al8(w * (NSEGP * SW))
    pltpu.sync_copy(parts, part_h.at[pl.ds(woff, NSEGP * SW)])


def _tc_body(hs_ref, ids_ref, am_ref, typ_ref, out_ref):
    b = pl.program_id(0)
    ids = ids_ref[0]
    am = am_ref[0]
    typ = typ_ref[0]
    z = (ids == 0).astype(jnp.float32)
    tri = (lax.broadcasted_iota(jnp.int32, (S, S), 0)
           <= lax.broadcasted_iota(jnp.int32, (S, S), 1)).astype(jnp.float32)
    cz = jnp.dot(z, tri, preferred_element_type=jnp.float32)
    valid = (cz == 0) & (ids != 101) & (ids != 102) & (am == 1)
    mterm = (valid & (typ == 0)).astype(jnp.float32)
    mtext = (valid & (typ == 1)).astype(jnp.float32)
    f = (b >= SB).astype(jnp.float32)
    hsum = (hs_ref[0, 0] + hs_ref[1, 0] + hs_ref[2, 0]
            + hs_ref[3, 0] * f)
    wt = jnp.sum(hsum * mterm.reshape(S, 1), axis=0)
    wx = jnp.sum(hsum * mtext.reshape(S, 1), axis=0)
    out_ref[0] = jnp.concatenate([wt[None, :], wx[None, :]], axis=0)


_tc_sums = pl.pallas_call(
    _tc_body,
    grid=(B,),
    in_specs=[
        pl.BlockSpec((NL, 1, S, D), lambda b: (0, b, 0, 0)),
        pl.BlockSpec((1, 1, S), lambda b: (b, 0, 0)),
        pl.BlockSpec((1, 1, S), lambda b: (b, 0, 0)),
        pl.BlockSpec((1, 1, S), lambda b: (b, 0, 0)),
    ],
    out_specs=pl.BlockSpec((1, 2, D), lambda b: (b, 0, 0)),
    out_shape=jax.ShapeDtypeStruct((B, 2, D), jnp.float32),
    compiler_params=pltpu.CompilerParams(
        dimension_semantics=("arbitrary",)),
)


def _combine_body(parts_ref, tc_ref, n2_ref, out_ref):
    psum = jnp.sum(parts_ref[...], axis=0)
    pad = jnp.zeros((2 * B - NSEGP, SW), jnp.float32)
    total = tc_ref[...] + jnp.concatenate([psum, pad], axis=0)
    out_ref[...] = total * (jnp.float32(0.25) / n2_ref[...])


_combine = pl.pallas_call(
    _combine_body,
    out_shape=jax.ShapeDtypeStruct((2 * B, D), jnp.float32),
)

_phase1 = functools.partial(
    pl.kernel,
    out_type=[jax.ShapeDtypeStruct((NW * NSEGP * SW,), jnp.float32),
              jax.ShapeDtypeStruct((B, LANES), jnp.int32)],
    mesh=_mesh,
    compiler_params=pltpu.CompilerParams(needs_layout_passes=False,
                                         use_tc_tiling_on_sc=False),
    scratch_types=[
        pltpu.VMEM((S,), jnp.int32),
        pltpu.VMEM((S,), jnp.int32),
        pltpu.VMEM((S,), jnp.int32),
        pltpu.VMEM((REG + LANES,), jnp.int32),
        pltpu.VMEM((S,), jnp.int32),
        pltpu.VMEM((IDXLEN,), jnp.int32),
        pltpu.VMEM((2, K, SW), jnp.float32),
        pltpu.VMEM((NSEGP * SW,), jnp.float32),
        pltpu.VMEM((B, LANES), jnp.int32),
        pltpu.VMEM((LANES,), jnp.int32),
        pltpu.SMEM((NCHMAX,), jnp.int32),
        pltpu.VMEM_SHARED((B, REG), jnp.int32),
        pltpu.VMEM_SHARED((B, LANES), jnp.int32),
        pltpu.SemaphoreType.DMA,
    ],
)(_p1_body)

@jax.jit
def kernel(bert_out, input_ids, attention_mask, token_type_ids):
    table = bert_out[-1].reshape(NROWS, SW)
    hs4 = bert_out[-NL:]
    ids32 = input_ids.astype(jnp.int32)
    am32 = attention_mask.astype(jnp.int32)
    typ32 = token_type_ids.astype(jnp.int32)
    part, cnt = _phase1(table, ids32, am32, typ32)
    tc = _tc_sums(hs4, ids32.reshape(B, 1, S), am32.reshape(B, 1, S),
                  typ32.reshape(B, 1, S))
    n2 = cnt[:, 0:2].reshape(2 * B, 1).astype(jnp.float32)
    out2 = _combine(part.reshape(NW, NSEGP, SW), tc.reshape(2 * B, D), n2)
    return out2.reshape(B, 2 * D)


# speedup vs baseline: 1.2093x; 1.0095x over previous
import functools

import jax
import jax.numpy as jnp
from jax import lax
from jax.experimental import pallas as pl
from jax.experimental.pallas import tpu as pltpu
from jax.experimental.pallas import tpu_sc as plsc

B, S, D = 16, 512, 768
NL = 4
SW = D
NV = SW // 16
NROWS = B * S
REG = 544
K = 8
IDXLEN = B * REG + 64
LANES = 16
NSEG = 2 * B
NW = 16
SB = 2
NSEGP = 2 * SB
NCHMAX = B * (REG // K + 2) + 8 * NW + 8

_mesh = plsc.VectorSubcoreMesh(core_axis_name="c", subcore_axis_name="s",
                               num_cores=1)


def _al8(x):
    return pl.multiple_of(x, 8)


def _lane_iota():
    return jnp.arange(LANES, dtype=jnp.int32)


def _p1_body(table, ids_h, am_h, typ_h, part_h, cnt_h,
             row_i, row_a, row_t, bld, msk, idxb, gbs, parts, cntv,
             cstage, wl, sh_idx, sh_cnt, sem0):
    ci = lax.axis_index("c")
    si = lax.axis_index("s")
    b = si
    iota = _lane_iota()
    zero16 = jnp.zeros((LANES,), jnp.int32)
    zf = jnp.zeros((LANES,), jnp.float32)

    pltpu.sync_copy(ids_h.at[b], row_i)
    pltpu.sync_copy(am_h.at[b], row_a)
    pltpu.sync_copy(typ_h.at[b], row_t)

    def _zb(i, _):
        bld[pl.ds(_al8(i * LANES), LANES)] = zero16
        return 0
    lax.fori_loop(0, (REG + LANES) // LANES, _zb, 0)

    c_b = b * S

    def _pass1(v, carry):
        zcarry, off, nt = carry
        o = _al8(v * LANES)
        iv = row_i[pl.ds(o, LANES)]
        av = row_a[pl.ds(o, LANES)]
        tv = row_t[pl.ds(o, LANES)]
        z = (iv == 0).astype(jnp.int32)
        cs = jnp.cumsum(z) + zcarry
        valid = (cs == 0) & (iv != 101) & (iv != 102) & (av == 1)
        mterm = valid & (tv == 0)
        mtext = valid & (tv == 1)
        msk[pl.ds(o, LANES)] = mtext.astype(jnp.int32)
        pos = iota + v * LANES + c_b
        mi = mterm.astype(jnp.int32)
        h = jnp.sum(mi)
        pr = jnp.cumsum(mi) - mi + off
        plsc.store_scatter(bld, [pr], pos, mask=mterm)
        return zcarry + jnp.sum(z), off + h, nt + h

    _, off, nt = lax.fori_loop(0, S // LANES, _pass1, (0, 0, 0))
    off = (off + K - 1) & (-K)

    def _pass2(v, carry):
        off, nx = carry
        o = _al8(v * LANES)
        mtext = msk[pl.ds(o, LANES)] != 0
        pos = iota + v * LANES + c_b
        mi = mtext.astype(jnp.int32)
        h = jnp.sum(mi)
        pr = jnp.cumsum(mi) - mi + off
        plsc.store_scatter(bld, [pr], pos, mask=mtext)
        return off + h, nx + h

    _, nx = lax.fori_loop(0, S // LANES, _pass2, (off, 0))

    cstage[...] = (jnp.where(iota == 0, nt, 0)
                   + jnp.where(iota == 1, nx, 0))
    pltpu.sync_copy(cstage, sh_cnt.at[b])
    pltpu.sync_copy(bld.at[pl.ds(0, REG)], sh_idx.at[b])

    @pl.when(ci == 0)
    def _():
        pltpu.sync_copy(cstage, cnt_h.at[b])

    plsc.subcore_barrier()

    pltpu.sync_copy(sh_cnt, cntv)
    for bb in range(SB):
        pltpu.sync_copy(sh_idx.at[bb], idxb.at[pl.ds(bb * REG, REG)])
    for i in range(64 // LANES):
        idxb[pl.ds(B * REG + i * LANES, LANES)] = zero16

    w = ci * 16 + si

    def _zp(i, _):
        parts[pl.ds(_al8(i * LANES), LANES)] = zf
        return 0
    lax.fori_loop(0, NSEGP * SW // LANES, _zp, 0)

    cc = 0
    for bb in range(SB):
        row = cntv[bb]
        ntb = row[0]
        nxb = row[1]
        t1s = (ntb + K - 1) & (-K)
        for t in range(2):
            stt = bb * REG + t1s * t
            ln = jnp.where(t == 0, ntb, nxb)
            seg = 2 * bb + t
            nch = (ln + K - 1) // K

            def _wl(j, cc, stt=stt, ln=ln, seg=seg):
                word = ((stt + j * K) | (seg << 15)
                        | (jnp.minimum(ln - j * K, 15) << 20))
                wl[cc] = word
                return cc + 1
            cc = lax.fori_loop(0, nch, _wl, cc)
    def _snt(i, _):
        wl[cc + i] = 0
        return 0
    lax.fori_loop(0, 8 * NW, _snt, 0)

    def _issue(word, buf, sem):
        o = pl.multiple_of(word & 32767, 8)
        return pltpu.async_copy(table.at[idxb.at[pl.ds(o, K)]], buf, sem)

    def _fold_into_parts(a, seg):
        for k in range(NV):
            plsc.addupdate(parts.at[pl.ds(_al8(seg * SW + k * LANES), LANES)],
                           a[k])

    def _acc_buf(buf, word):
        lnr = (word >> 20) & 15
        seg = (word >> 15) & 31
        a = [zf] * NV
        for j in range(K):
            c = j < lnr
            for k in range(NV):
                r = buf[j, pl.ds(k * LANES, LANES)]
                a[k] = a[k] + jnp.where(c, r, zf)
        _fold_into_parts(a, seg)

    q = (((cc + NW - 1) // NW) + 1) & (-2)
    base = w * q
    n2 = q // 2

    def _duo(i2, _):
        b2 = base + i2 * 2
        cps = []
        for u in range(2):
            cps.append(_issue(wl[b2 + u], gbs.at[u], sem0))
        for u in range(2):
            cps[u].wait()
            _acc_buf(gbs.at[u], wl[b2 + u])
        return 0

    lax.fori_loop(0, n2, _duo, 0)

    woff = _al8(w * (NSEGP * SW))
    pltpu.sync_copy(parts, part_h.at[pl.ds(woff, NSEGP * SW)])


def _tc_body(hs_ref, ids_ref, am_ref, typ_ref, out_ref):
    b = pl.program_id(0)
    ids = ids_ref[0]
    am = am_ref[0]
    typ = typ_ref[0]
    z = (ids == 0).astype(jnp.float32)
    tri = (lax.broadcasted_iota(jnp.int32, (S, S), 0)
           <= lax.broadcasted_iota(jnp.int32, (S, S), 1)).astype(jnp.float32)
    cz = jnp.dot(z, tri, preferred_element_type=jnp.float32)
    valid = (cz == 0) & (ids != 101) & (ids != 102) & (am == 1)
    mterm = (valid & (typ == 0)).astype(jnp.float32)
    mtext = (valid & (typ == 1)).astype(jnp.float32)
    f = (b >= SB).astype(jnp.float32)
    hsum = (hs_ref[0, 0] + hs_ref[1, 0] + hs_ref[2, 0]
            + hs_ref[3, 0] * f)
    wt = jnp.sum(hsum * mterm.reshape(S, 1), axis=0)
    wx = jnp.sum(hsum * mtext.reshape(S, 1), axis=0)
    out_ref[0] = jnp.concatenate([wt[None, :], wx[None, :]], axis=0)


_tc_sums = pl.pallas_call(
    _tc_body,
    grid=(B,),
    in_specs=[
        pl.BlockSpec((NL, 1, S, D), lambda b: (0, b, 0, 0)),
        pl.BlockSpec((1, 1, S), lambda b: (b, 0, 0)),
        pl.BlockSpec((1, 1, S), lambda b: (b, 0, 0)),
        pl.BlockSpec((1, 1, S), lambda b: (b, 0, 0)),
    ],
    out_specs=pl.BlockSpec((1, 2, D), lambda b: (b, 0, 0)),
    out_shape=jax.ShapeDtypeStruct((B, 2, D), jnp.float32),
    compiler_params=pltpu.CompilerParams(
        dimension_semantics=("arbitrary",)),
)


def _combine_body(parts_ref, tc_ref, n2_ref, out_ref):
    psum = jnp.sum(parts_ref[...], axis=0)
    pad = jnp.zeros((2 * B - NSEGP, SW), jnp.float32)
    total = tc_ref[...] + jnp.concatenate([psum, pad], axis=0)
    out_ref[...] = total * (jnp.float32(0.25) / n2_ref[...])


_combine = pl.pallas_call(
    _combine_body,
    out_shape=jax.ShapeDtypeStruct((2 * B, D), jnp.float32),
)

_phase1 = functools.partial(
    pl.kernel,
    out_type=[jax.ShapeDtypeStruct((NW * NSEGP * SW,), jnp.float32),
              jax.ShapeDtypeStruct((B, LANES), jnp.int32)],
    mesh=_mesh,
    compiler_params=pltpu.CompilerParams(needs_layout_passes=False,
                                         use_tc_tiling_on_sc=False),
    scratch_types=[
        pltpu.VMEM((S,), jnp.int32),
        pltpu.VMEM((S,), jnp.int32),
        pltpu.VMEM((S,), jnp.int32),
        pltpu.VMEM((REG + LANES,), jnp.int32),
        pltpu.VMEM((S,), jnp.int32),
        pltpu.VMEM((IDXLEN,), jnp.int32),
        pltpu.VMEM((2, K, SW), jnp.float32),
        pltpu.VMEM((NSEGP * SW,), jnp.float32),
        pltpu.VMEM((B, LANES), jnp.int32),
        pltpu.VMEM((LANES,), jnp.int32),
        pltpu.SMEM((NCHMAX,), jnp.int32),
        pltpu.VMEM_SHARED((B, REG), jnp.int32),
        pltpu.VMEM_SHARED((B, LANES), jnp.int32),
        pltpu.SemaphoreType.DMA,
    ],
)(_p1_body)

@jax.jit
def kernel(bert_out, input_ids, attention_mask, token_type_ids):
    table = bert_out[-1].reshape(NROWS, SW)
    hs4 = bert_out[-NL:]
    ids32 = input_ids.astype(jnp.int32)
    am32 = attention_mask.astype(jnp.int32)
    typ32 = token_type_ids.astype(jnp.int32)
    part, cnt = _phase1(table, ids32, am32, typ32)
    tc = _tc_sums(hs4, ids32.reshape(B, 1, S), am32.reshape(B, 1, S),
                  typ32.reshape(B, 1, S))
    n2 = cnt[:, 0:2].reshape(2 * B, 1).astype(jnp.float32)
    out2 = _combine(part.reshape(NW, NSEGP, SW), tc.reshape(2 * B, D), n2)
    return out2.reshape(B, 2 * D)
